# Initial kernel scaffold; baseline (speedup 1.0000x reference)
import jax
import jax.numpy as jnp
from jax.experimental import pallas as pl


def kernel(z, edge_index, edge_attr, params):
    raise NotImplementedError("write your pallas kernel here")


# speedup vs baseline: 1.3475x; 1.3475x over previous
import functools

import jax
import jax.numpy as jnp
from jax import lax
from jax.experimental import pallas as pl
from jax.experimental.pallas import tpu as pltpu, tpu_sc as plsc

N_NODES_C = 10000
N_EDGES_C = 320000
NPAD = 10000
D_X = 128
D_EA = 16
D_MSG = 144
N_EXP = 4

EB = 2000
N_EBLK = N_EDGES_C // EB

_F32 = jnp.float32
_HIGH = jax.lax.Precision.HIGHEST


def _dg(a, b, dims):
    return lax.dot_general(a, b, (dims, ((), ())),
                           preferred_element_type=_F32, precision=_HIGH)


def _leaky(x, slope):
    return jnp.where(x >= 0, x, slope * x)


def _stats_body(ea_ref, st_ref):
    ea = ea_ref[...]
    s2 = _dg(ea, ea, ((0,), (0,)))
    s1 = jnp.sum(ea, axis=0)
    blk = jnp.concatenate([s2, s1[None, :], jnp.zeros((7, 16), _F32)], axis=0)

    @pl.when(pl.program_id(0) == 0)
    def _():
        st_ref[...] = jnp.zeros_like(st_ref)

    st_ref[...] += blk


def _stats_call(edge_attr):
    return pl.pallas_call(
        _stats_body,
        grid=(N_EBLK,),
        in_specs=[pl.BlockSpec((EB, D_EA), lambda i: (i, 0))],
        out_specs=pl.BlockSpec((24, 16), lambda i: (0, 0)),
        out_shape=jax.ShapeDtypeStruct((24, 16), _F32),
    )(edge_attr)


def _edge_body(ea_ref, w1r_ref, b1r_ref, w2r_ref, b2r_ref,
               w1e_ref, b1e_ref, w2e_ref, b2e_ref,
               routeT_ref, eaout_ref, lb_ref):
    ea = ea_ref[...]
    hr = _leaky(_dg(ea, w1r_ref[...], ((1,), (1,))) + b1r_ref[...], 0.01)
    logits = _dg(hr, w2r_ref[...], ((1,), (1,))) + b2r_ref[...]
    m = jnp.max(logits, axis=1, keepdims=True)
    ex = jnp.exp(logits - m)
    route = ex / jnp.sum(ex, axis=1, keepdims=True)
    routeT_ref[...] = route

    he = _leaky(_dg(ea, w1e_ref[...], ((1,), (1,))) + b1e_ref[...], 0.01)
    eaout_ref[...] = _dg(he, w2e_ref[...], ((1,), (1,))) + b2e_ref[...]

    colsum = jnp.sum(route, axis=0)
    sq = jnp.sum(route * route)
    row = jnp.concatenate([colsum, sq[None], jnp.zeros((123,), _F32)])
    rowm = lax.broadcasted_iota(jnp.int32, (8, 128), 0) == 0
    blk = jnp.where(rowm, row[None, :], 0.0)

    @pl.when(pl.program_id(0) == 0)
    def _():
        lb_ref[...] = jnp.zeros_like(lb_ref)

    lb_ref[...] += blk


def _edge_call(edge_attr, w1r, b1r, w2r, b2r, w1e, b1e, w2e, b2e):
    zero = lambda i: (0, 0)
    return pl.pallas_call(
        _edge_body,
        grid=(N_EBLK,),
        in_specs=[
            pl.BlockSpec((EB, D_EA), lambda i: (i, 0)),
            pl.BlockSpec(w1r.shape, zero), pl.BlockSpec(b1r.shape, zero),
            pl.BlockSpec(w2r.shape, zero), pl.BlockSpec(b2r.shape, zero),
            pl.BlockSpec(w1e.shape, zero), pl.BlockSpec(b1e.shape, zero),
            pl.BlockSpec(w2e.shape, zero), pl.BlockSpec(b2e.shape, zero),
        ],
        out_specs=[
            pl.BlockSpec((EB, N_EXP), lambda i: (i, 0)),
            pl.BlockSpec((EB, D_EA), lambda i: (i, 0)),
            pl.BlockSpec((8, 128), zero),
        ],
        out_shape=[
            jax.ShapeDtypeStruct((N_EDGES_C, N_EXP), _F32),
            jax.ShapeDtypeStruct((N_EDGES_C, D_EA), _F32),
            jax.ShapeDtypeStruct((8, 128), _F32),
        ],
    )(edge_attr, w1r, b1r, w2r, b2r, w1e, b1e, w2e, b2e)


EMB_B = 2000


def _emb_body(z_ref, emb_ref, xo_ref):
    z = z_ref[...]
    ids = lax.broadcasted_iota(jnp.int32, (EMB_B, 200), 1)
    oh = (z == ids).astype(_F32)
    xo_ref[...] = _dg(oh, emb_ref[...], ((1,), (0,)))


def _emb_call(z, emb):
    return pl.pallas_call(
        _emb_body,
        grid=(N_NODES_C // EMB_B,),
        in_specs=[
            pl.BlockSpec((EMB_B, 1), lambda i: (i, 0)),
            pl.BlockSpec((200, D_X), lambda i: (0, 0)),
        ],
        out_specs=pl.BlockSpec((EMB_B, D_X), lambda i: (i, 0)),
        out_shape=jax.ShapeDtypeStruct((N_NODES_C, D_X), _F32),
    )(z.reshape(N_NODES_C, 1), emb)


NB = 2000
N_NBLK = N_NODES_C // NB


def _stagec1_body(hx_ref, hea_ref, w1_ref, b1_ref, g1_ref, bt1_ref,
                  w2_ref, b2_ref, out_ref, st_ref):
    def mlp(hx, hea, k, slope):
        w1 = w1_ref[k]
        y = (_dg(hx, w1[:, :D_X], ((1,), (1,)))
             + _dg(hea, w1[:, D_X:], ((1,), (1,))) + b1_ref[k][None, :])
        mu = jnp.mean(y, axis=-1, keepdims=True)
        var = jnp.mean((y - mu) ** 2, axis=-1, keepdims=True)
        y = (y - mu) / jnp.sqrt(var + 1e-5) * g1_ref[k][None, :] + bt1_ref[k][None, :]
        y = _leaky(y, slope)
        return _dg(y, w2_ref[k], ((1,), (1,))) + b2_ref[k][None, :]

    hxs = [hx_ref[e] for e in range(N_EXP)]
    heas = hea_ref[0] + hea_ref[1]
    heal = [heas[:, e * D_EA:(e + 1) * D_EA] for e in range(N_EXP)]
    h0x = hxs[0] + hxs[1] + hxs[2] + hxs[3]
    h0ea = heal[0] + heal[1] + heal[2] + heal[3]
    out = mlp(h0x, h0ea, 0, 0.01)
    for e in range(N_EXP):
        out = out + mlp(hxs[e], heal[e], 1 + e, 0.0)
    out_ref[...] = out

    s1 = jnp.sum(out, axis=0)
    s2 = jnp.sum(out * out, axis=0)
    rid = lax.broadcasted_iota(jnp.int32, (8, 128), 0)
    blk = jnp.where(rid == 0, s1[None, :],
                    jnp.where(rid == 1, s2[None, :], 0.0))

    @pl.when(pl.program_id(0) == 0)
    def _():
        st_ref[...] = jnp.zeros_like(st_ref)

    st_ref[...] += blk


def _stagec1_call(hx, hea, w1, b1, g1, bt1, w2, b2):
    zero = lambda i: (0, 0)
    zero3 = lambda i: (0, 0, 0)
    return pl.pallas_call(
        _stagec1_body,
        grid=(N_NBLK,),
        in_specs=[
            pl.BlockSpec((N_EXP, NB, D_X), lambda i: (0, i, 0)),
            pl.BlockSpec((2, NB, D_X), lambda i: (0, i, 0)),
            pl.BlockSpec(w1.shape, zero3), pl.BlockSpec(b1.shape, zero),
            pl.BlockSpec(g1.shape, zero), pl.BlockSpec(bt1.shape, zero),
            pl.BlockSpec(w2.shape, zero3), pl.BlockSpec(b2.shape, zero),
        ],
        out_specs=[
            pl.BlockSpec((NB, D_X), lambda i: (i, 0)),
            pl.BlockSpec((8, 128), zero),
        ],
        out_shape=[
            jax.ShapeDtypeStruct((N_NODES_C, D_X), _F32),
            jax.ShapeDtypeStruct((8, 128), _F32),
        ],
    )(hx, hea, w1, b1, g1, bt1, w2, b2)


def _stagec2_body(out_ref, x_ref, st_ref, nw_ref, nb_ref, ms_ref, al_ref, xo_ref):
    inv_n = 1.0 / N_NODES_C
    mean = st_ref[0, :][None, :] * inv_n
    e2 = st_ref[1, :][None, :] * inv_n
    ms = ms_ref[...]
    var = e2 - (2.0 * ms * mean - ms * ms * mean) * mean
    o2 = out_ref[...] - ms * mean
    gn = o2 / jnp.sqrt(var + 1e-5) * nw_ref[...] + nb_ref[...]
    xo_ref[...] = x_ref[...] + al_ref[0, 0] * gn


def _stagec2_call(out, x, st, nw, nb, ms, al):
    zero = lambda i: (0, 0)
    return pl.pallas_call(
        _stagec2_body,
        grid=(N_NBLK,),
        in_specs=[
            pl.BlockSpec((NB, D_X), lambda i: (i, 0)),
            pl.BlockSpec((NB, D_X), lambda i: (i, 0)),
            pl.BlockSpec((8, 128), zero),
            pl.BlockSpec(nw.shape, zero), pl.BlockSpec(nb.shape, zero),
            pl.BlockSpec(ms.shape, zero), pl.BlockSpec(al.shape, zero),
        ],
        out_specs=pl.BlockSpec((NB, D_X), lambda i: (i, 0)),
        out_shape=jax.ShapeDtypeStruct((N_NODES_C, D_X), _F32),
    )(out, x, st, nw, nb, ms, al)


SC_B = 80
EPT = N_EDGES_C // 16
SC_NBLK = EPT // SC_B
EPT_EA = N_EDGES_C // 32
SC_NBLK_EA = EPT_EA // SC_B
ROWS_PT = NPAD // 16
ZCH = 25
ROWS_OUT = 1000


def _sc_body(x_hbm, src_hbm, dst_hbm, routeT_hbm, ea_hbm, outx_hbm, outea_hbm,
             acc, srcv, dstv, wv, w4, eav, xg, msg, zbuf, wrep, wrep4, sem):
    c = lax.axis_index("c")
    s = lax.axis_index("s")

    def zrow(r, _):
        for k in range(D_X // 16):
            zbuf[r, k * 16:(k + 1) * 16] = jnp.zeros((16,), _F32)
        return 0

    lax.fori_loop(0, ZCH, zrow, 0)

    def zero_acc():
        def zcp(t, _):
            pltpu.sync_copy(zbuf, acc.at[pl.ds(s * ROWS_PT + t * ZCH, ZCH), :])
            return 0

        lax.fori_loop(0, ROWS_PT // ZCH, zcp, 0)

    for p in range(2):
        e = 2 * c + p
        zero_acc()
        plsc.subcore_barrier()

        def blk(j, _):
            base = s * EPT + j * SC_B
            pltpu.sync_copy(src_hbm.at[pl.ds(base, SC_B)], srcv)
            pltpu.sync_copy(dst_hbm.at[pl.ds(base, SC_B)], dstv)
            woff = pl.multiple_of(e * N_EDGES_C + base, 8)
            pltpu.sync_copy(routeT_hbm.at[pl.ds(woff, SC_B)], wv)
            pltpu.async_copy(x_hbm.at[srcv], xg, sem).wait()

            def per_grp(g, _):
                wvec = wv[pl.ds(g * 16, 16)]
                for jj in range(16):
                    wrep[jj, :] = jnp.broadcast_to(wvec[jj], (16,))

                def inner(jj, _):
                    i = g * 16 + jj
                    wr = wrep[jj, :]
                    for k in range(D_X // 16):
                        msg[i, k * 16:(k + 1) * 16] = xg[i, k * 16:(k + 1) * 16] * wr
                    return 0

                lax.fori_loop(0, 16, inner, 0)
                return 0

            lax.fori_loop(0, SC_B // 16, per_grp, 0)
            pltpu.sync_copy(msg, acc.at[dstv], add=True)
            return 0

        lax.fori_loop(0, SC_NBLK, blk, 0)
        plsc.subcore_barrier()

        @pl.when(s < 10)
        def _():
            pltpu.sync_copy(acc.at[pl.ds(s * ROWS_OUT, ROWS_OUT), :],
                            outx_hbm.at[e, pl.ds(s * ROWS_OUT, ROWS_OUT), :])

        plsc.subcore_barrier()

    zero_acc()

    def zmsg(r, _):
        for k in range(4):
            msg[r, 64 + k * 16: 64 + (k + 1) * 16] = jnp.zeros((16,), _F32)
        return 0

    lax.fori_loop(0, SC_B, zmsg, 0)
    plsc.subcore_barrier()

    def blk_ea(j, _):
        base = (c * 16 + s) * EPT_EA + j * SC_B
        pltpu.sync_copy(dst_hbm.at[pl.ds(base, SC_B)], dstv)
        pltpu.sync_copy(ea_hbm.at[pl.ds(base, SC_B), :], eav)
        for e4 in range(N_EXP):
            woff = pl.multiple_of(e4 * N_EDGES_C + base, 8)
            pltpu.sync_copy(routeT_hbm.at[pl.ds(woff, SC_B)], w4.at[e4])

        def per_grp(g, _):
            for e4 in range(N_EXP):
                wvec = w4[e4, pl.ds(g * 16, 16)]
                for jj in range(16):
                    wrep4[jj, e4 * 16:(e4 + 1) * 16] = jnp.broadcast_to(wvec[jj], (16,))

            def inner(jj, _):
                i = g * 16 + jj
                ear = eav[i, :]
                for e4 in range(N_EXP):
                    msg[i, e4 * 16:(e4 + 1) * 16] = ear * wrep4[jj, e4 * 16:(e4 + 1) * 16]
                return 0

            lax.fori_loop(0, 16, inner, 0)
            return 0

        lax.fori_loop(0, SC_B // 16, per_grp, 0)
        pltpu.sync_copy(msg, acc.at[dstv], add=True)
        return 0

    lax.fori_loop(0, SC_NBLK_EA, blk_ea, 0)
    plsc.subcore_barrier()

    @pl.when(s < 10)
    def _():
        pltpu.sync_copy(acc.at[pl.ds(s * ROWS_OUT, ROWS_OUT), :],
                        outea_hbm.at[c, pl.ds(s * ROWS_OUT, ROWS_OUT), :])


@functools.cache
def _sc_call_build():
    return functools.partial(
        pl.kernel,
        out_type=(jax.ShapeDtypeStruct((N_EXP, NPAD, D_X), _F32),
                  jax.ShapeDtypeStruct((2, NPAD, D_X), _F32)),
        mesh=plsc.VectorSubcoreMesh(core_axis_name="c", subcore_axis_name="s"),
        scratch_types=[
            pltpu.VMEM_SHARED((NPAD, D_X), _F32),
            pltpu.VMEM((SC_B,), jnp.int32),
            pltpu.VMEM((SC_B,), jnp.int32),
            pltpu.VMEM((SC_B,), _F32),
            pltpu.VMEM((N_EXP, SC_B), _F32),
            pltpu.VMEM((SC_B, D_EA), _F32),
            pltpu.VMEM((SC_B, D_X), _F32),
            pltpu.VMEM((SC_B, D_X), _F32),
            pltpu.VMEM((ZCH, D_X), _F32),
            pltpu.VMEM((16, 16), _F32),
            pltpu.VMEM((16, 64), _F32),
            pltpu.SemaphoreType.DMA,
        ],
    )(_sc_body)


def _segment_stage(x, src, dst, routeT, eaout):
    return _sc_call_build()(x, src, dst, routeT.reshape(-1), eaout)


def _fold(layer, mu, cov):
    W, b, g, bt = layer["W"], layer["b"], layer["gamma"], layer["beta"]
    mh = W @ mu + b
    vh = jnp.einsum("ij,jk,ik->i", W, cov, W)
    sc = g / jnp.sqrt(vh + 1e-5)
    return W * sc[:, None], (b - mh) * sc + bt


def _lb_finalize(lb):
    n = float(N_EDGES_C)
    mean_r = lb[0, :4] / n
    sq_mean = lb[0, 4] / n
    frac = 1.0 / N_EXP
    ebl = (jnp.sum(mean_r ** 2) - frac) * (1.0 / (1.0 - frac))
    ul = 1.0 - sq_mean
    a = b_ = 0.1
    t = (a + b_) * 0.1
    unf = a * ebl + b_ * ul
    return (jnp.maximum(unf, t) - t) * ((a + b_) / (a + b_ - t))


def kernel(z, edge_index, edge_attr, params):
    z = z.astype(jnp.int32)
    src = edge_index[0].astype(jnp.int32)
    dst = edge_index[1].astype(jnp.int32)

    stats = _stats_call(edge_attr)
    mu = stats[16] / N_EDGES_C
    cov = stats[:16] / N_EDGES_C - jnp.outer(mu, mu)

    x = _emb_call(z, params["emb"])

    total = jnp.asarray(0.0, _F32)
    for i in range(2):
        cp = params["convs"][i]
        w1r, b1r = _fold(cp["router"][0], mu, cov)
        w1e, b1e = _fold(cp["edge"][0], mu, cov)
        route, eaout, lb = _edge_call(
            edge_attr,
            w1r, b1r[None, :],
            cp["router"][1]["W"], cp["router"][1]["b"][None, :],
            w1e, b1e[None, :],
            cp["edge"][1]["W"], cp["edge"][1]["b"][None, :],
        )
        routeT = route.T
        total = total + _lb_finalize(lb)

        hx, hea = _segment_stage(x, src, dst, routeT, eaout)

        mlps = [cp["shared"]] + list(cp["experts"])
        w1 = jnp.stack([m[0]["W"] for m in mlps])
        b1 = jnp.stack([m[0]["b"] for m in mlps])
        g1 = jnp.stack([m[0]["gamma"] for m in mlps])
        bt1 = jnp.stack([m[0]["beta"] for m in mlps])
        w2 = jnp.stack([m[1]["W"] for m in mlps])
        b2 = jnp.stack([m[1]["b"] for m in mlps])
        nm = params["norms"][i]
        out, st = _stagec1_call(hx, hea, w1, b1, g1, bt1, w2, b2)
        x = _stagec2_call(
            out, x, st,
            nm["weight"][None, :], nm["bias"][None, :],
            nm["mean_scale"][None, :],
            params["alpha"][i].reshape(1, 1),
        )

    return x, total
